# Initial kernel scaffold; baseline (speedup 1.0000x reference)
import jax
import jax.numpy as jnp
from jax.experimental import pallas as pl


def kernel(x, neighbour_h, neighbour_c, mask, W_iou, b_iou, W_fin, b_fin, W_fg, b_fg, W_aggr, b_aggr):
    raise NotImplementedError("write your pallas kernel here")


# speedup vs baseline: 3.0201x; 3.0201x over previous
import functools

import jax
import jax.numpy as jnp
from jax.experimental import pallas as pl
from jax.experimental.pallas import tpu as pltpu


def _tree_lstm_block(hs, nch, x_ref, nh_ref, nc_ref, m_ref,
                     wx_ref, bx_ref, wfg_ref, bfg_ref, waggr_ref, baggr_ref,
                     h_ref, c_ref):
    xw = jnp.dot(x_ref[...], wx_ref[...], preferred_element_type=jnp.float32)
    xw = (xw + bx_ref[...]) * m_ref[...]
    iou_input = xw[:, : 3 * hs]
    f_input = xw[:, 3 * hs:]

    nh = nh_ref[...]
    h_sum = jnp.sum(nh, axis=1)
    iou_aggr = jnp.dot(h_sum, waggr_ref[...],
                       preferred_element_type=jnp.float32) + baggr_ref[...]

    c_aggr = None
    for ch in range(nch):
        fg = jnp.dot(nh[:, ch, :], wfg_ref[...],
                     preferred_element_type=jnp.float32) + bfg_ref[...]
        f = jax.nn.sigmoid(fg + f_input)
        contrib = f * nc_ref[:, ch, :]
        c_aggr = contrib if c_aggr is None else c_aggr + contrib

    iou = iou_input + iou_aggr
    i = jax.nn.sigmoid(iou[:, :hs])
    o = jax.nn.sigmoid(iou[:, hs: 2 * hs])
    u = jnp.tanh(iou[:, 2 * hs:])
    c = i * u + c_aggr
    h_ref[...] = o * jnp.tanh(c)
    c_ref[...] = c


def kernel(x, neighbour_h, neighbour_c, mask,
           W_iou, b_iou, W_fin, b_fin, W_fg, b_fg, W_aggr, b_aggr):
    n, xs = x.shape
    _, nch, hs = neighbour_h.shape

    bn = 1000
    if n % bn:
        bn = 8
    grid = (n // bn,)

    maskf = mask.astype(jnp.float32).reshape(n, 1)
    wx = jnp.concatenate([W_iou, W_fin], axis=1)
    bx = jnp.concatenate([b_iou, b_fin]).reshape(1, 4 * hs)
    bfg = b_fg.reshape(1, hs)
    baggr = b_aggr.reshape(1, 3 * hs)

    row = lambda i: (i, 0)
    row3 = lambda i: (i, 0, 0)
    rep2 = lambda i: (0, 0)

    h_out, c_out = pl.pallas_call(
        functools.partial(_tree_lstm_block, hs, nch),
        grid=grid,
        in_specs=[
            pl.BlockSpec((bn, xs), row),
            pl.BlockSpec((bn, nch, hs), row3),
            pl.BlockSpec((bn, nch, hs), row3),
            pl.BlockSpec((bn, 1), row),
            pl.BlockSpec((xs, 4 * hs), rep2),
            pl.BlockSpec((1, 4 * hs), rep2),
            pl.BlockSpec((hs, hs), rep2),
            pl.BlockSpec((1, hs), rep2),
            pl.BlockSpec((hs, 3 * hs), rep2),
            pl.BlockSpec((1, 3 * hs), rep2),
        ],
        out_specs=[
            pl.BlockSpec((bn, hs), row),
            pl.BlockSpec((bn, hs), row),
        ],
        out_shape=[
            jax.ShapeDtypeStruct((n, hs), jnp.float32),
            jax.ShapeDtypeStruct((n, hs), jnp.float32),
        ],
        compiler_params=pltpu.CompilerParams(
            dimension_semantics=("arbitrary",),
        ),
    )(x, neighbour_h, neighbour_c, maskf, wx, bx, W_fg, bfg, W_aggr, baggr)
    return h_out, c_out
